# Initial kernel scaffold; baseline (speedup 1.0000x reference)
import jax
import jax.numpy as jnp
from jax.experimental import pallas as pl


def kernel(feat, edge_index, W1, b1, W2, b2):
    raise NotImplementedError("write your pallas kernel here")


# speedup vs baseline: 6.1602x; 6.1602x over previous
import functools

import jax
import jax.numpy as jnp
from jax import lax
from jax.experimental import pallas as pl
from jax.experimental.pallas import tpu as pltpu
from jax.experimental.pallas import tpu_sc as plsc

N = 10000
E = 320000
D_IN = 128
D_HID = 16
D_OUT = 128

NC = 2
NS = 16
NW = NC * NS
EPW = E // NW
CH = 80
NCH = EPW // CH
RPT = N // NS

_mesh = plsc.VectorSubcoreMesh(core_axis_name="c", subcore_axis_name="s")
_sc_params = pltpu.CompilerParams(use_tc_tiling_on_sc=False)


def _zero_fill(buf, nrows):
    zv = jnp.zeros((16,), jnp.float32)

    def body(i, _):
        buf[i] = zv
        return 0

    lax.fori_loop(0, nrows, body, 0)


def _ones_fill(buf, nrows):
    ov = jnp.ones((16,), jnp.float32)

    def body(i, _):
        buf[i] = ov
        return 0

    lax.fori_loop(0, nrows, body, 0)


@functools.partial(
    pl.kernel,
    mesh=_mesh,
    compiler_params=_sc_params,
    out_type=jax.ShapeDtypeStruct((NC, N, 16), jnp.float32),
    scratch_types=[
        pltpu.VMEM((CH,), jnp.int32),
        pltpu.VMEM((CH, 16), jnp.float32),
        pltpu.VMEM((RPT, 16), jnp.float32),
        pltpu.VMEM_SHARED((N, 16), jnp.float32),
    ],
)
def _sc_hist_src(src_hbm, out_hbm, idx_v, ones_v, stage_v, acc_sh):
    c = lax.axis_index("c")
    s = lax.axis_index("s")
    wid = s * NC + c
    _zero_fill(stage_v, RPT)
    _ones_fill(ones_v, CH)
    pltpu.sync_copy(stage_v, acc_sh.at[pl.ds(s * RPT, RPT)])
    plsc.subcore_barrier()
    ebase = wid * EPW

    def chunk(j, _):
        off = ebase + j * CH
        pltpu.sync_copy(src_hbm.at[pl.ds(off, CH)], idx_v)
        pltpu.sync_copy(ones_v, acc_sh.at[idx_v], add=True)
        return 0

    lax.fori_loop(0, NCH, chunk, 0)
    plsc.subcore_barrier()
    pltpu.sync_copy(acc_sh.at[pl.ds(s * RPT, RPT)], stage_v)
    pltpu.sync_copy(stage_v, out_hbm.at[c, pl.ds(s * RPT, RPT)])


def _make_sc_seg(with_hist):
    out_type = [jax.ShapeDtypeStruct((NC, N, 16), jnp.float32)]
    scratch = [
        pltpu.VMEM((CH,), jnp.int32),
        pltpu.VMEM((CH,), jnp.int32),
        pltpu.VMEM((CH, 16), jnp.float32),
        pltpu.VMEM((RPT, 16), jnp.float32),
        pltpu.VMEM_SHARED((N, 16), jnp.float32),
        pltpu.SemaphoreType.DMA,
    ]
    if with_hist:
        out_type = out_type + [jax.ShapeDtypeStruct((NC, N, 16), jnp.float32)]
        scratch = scratch + [
            pltpu.VMEM((CH, 16), jnp.float32),
            pltpu.VMEM_SHARED((N, 16), jnp.float32),
        ]

    @functools.partial(
        pl.kernel,
        mesh=_mesh,
        compiler_params=_sc_params,
        out_type=out_type,
        scratch_types=scratch,
    )
    def _seg(table_hbm, src_hbm, dst_hbm, *refs):
        if with_hist:
            (out_hbm, hist_hbm, src_v, dst_v, rows_v, stage_v, acc_sh, sem,
             ones_v, hacc_sh) = refs
        else:
            out_hbm, src_v, dst_v, rows_v, stage_v, acc_sh, sem = refs
        c = lax.axis_index("c")
        s = lax.axis_index("s")
        wid = s * NC + c
        _zero_fill(stage_v, RPT)
        pltpu.sync_copy(stage_v, acc_sh.at[pl.ds(s * RPT, RPT)])
        if with_hist:
            _ones_fill(ones_v, CH)
            pltpu.sync_copy(stage_v, hacc_sh.at[pl.ds(s * RPT, RPT)])
        plsc.subcore_barrier()
        ebase = wid * EPW

        def chunk(j, _):
            off = ebase + j * CH
            pltpu.sync_copy(src_hbm.at[pl.ds(off, CH)], src_v)
            pltpu.sync_copy(dst_hbm.at[pl.ds(off, CH)], dst_v)
            pltpu.async_copy(table_hbm.at[src_v], rows_v, sem).wait()
            pltpu.sync_copy(rows_v, acc_sh.at[dst_v], add=True)
            if with_hist:
                pltpu.sync_copy(ones_v, hacc_sh.at[dst_v], add=True)
            return 0

        lax.fori_loop(0, NCH, chunk, 0)
        plsc.subcore_barrier()
        pltpu.sync_copy(acc_sh.at[pl.ds(s * RPT, RPT)], stage_v)
        pltpu.sync_copy(stage_v, out_hbm.at[c, pl.ds(s * RPT, RPT)])
        if with_hist:
            pltpu.sync_copy(hacc_sh.at[pl.ds(s * RPT, RPT)], stage_v)
            pltpu.sync_copy(stage_v, hist_hbm.at[c, pl.ds(s * RPT, RPT)])

    return _seg


_sc_seg_hist = _make_sc_seg(True)
_sc_seg = _make_sc_seg(False)


RB = 400


def _norm(deg_parts):
    deg = deg_parts[0] + deg_parts[1]
    return lax.rsqrt(jnp.maximum(deg, 1.0))


def _tca_body(ds_ref, feat_ref, w1_ref, out_ref):
    nrm = _norm(ds_ref[...])[:, :1]
    out_ref[...] = jnp.dot(feat_ref[...] * nrm, w1_ref[...],
                           preferred_element_type=jnp.float32)


def _tcb_body(a_ref, dd_ref, ds_ref, b1_ref, out_ref):
    agg = (a_ref[0] + a_ref[1]) * _norm(dd_ref[...])
    z = jnp.maximum(agg + b1_ref[...], 0.0)
    out_ref[...] = z * _norm(ds_ref[...])


def _tcc_body(a_ref, dd_ref, w2_ref, b2_ref, out_ref):
    agg = (a_ref[0] + a_ref[1]) * _norm(dd_ref[...])
    out_ref[...] = jnp.dot(agg, w2_ref[...],
                           preferred_element_type=jnp.float32) + b2_ref[...]


_GRID = N // RB
_bs_parts = pl.BlockSpec((NC, RB, 16), lambda i: (0, i, 0))
_bs_h16 = pl.BlockSpec((RB, 16), lambda i: (i, 0))

_tca = pl.pallas_call(
    _tca_body,
    grid=(_GRID,),
    in_specs=[
        _bs_parts,
        pl.BlockSpec((RB, D_IN), lambda i: (i, 0)),
        pl.BlockSpec((D_IN, D_HID), lambda i: (0, 0)),
    ],
    out_specs=_bs_h16,
    out_shape=jax.ShapeDtypeStruct((N, D_HID), jnp.float32),
)

_tcb = pl.pallas_call(
    _tcb_body,
    grid=(_GRID,),
    in_specs=[
        _bs_parts,
        _bs_parts,
        _bs_parts,
        pl.BlockSpec((1, D_HID), lambda i: (0, 0)),
    ],
    out_specs=_bs_h16,
    out_shape=jax.ShapeDtypeStruct((N, D_HID), jnp.float32),
)

_tcc = pl.pallas_call(
    _tcc_body,
    grid=(_GRID,),
    in_specs=[
        _bs_parts,
        _bs_parts,
        pl.BlockSpec((D_HID, D_OUT), lambda i: (0, 0)),
        pl.BlockSpec((1, D_OUT), lambda i: (0, 0)),
    ],
    out_specs=pl.BlockSpec((RB, D_OUT), lambda i: (i, 0)),
    out_shape=jax.ShapeDtypeStruct((N, D_OUT), jnp.float32),
)


@jax.jit
def kernel(feat, edge_index, W1, b1, W2, b2):
    src = edge_index[0]
    dst = edge_index[1]
    deg_src = _sc_hist_src(src)
    h1 = _tca(deg_src, feat, W1)
    agg1, deg_dst = _sc_seg_hist(h1, src, dst)
    z = _tcb(agg1, deg_dst, deg_src, b1.reshape(1, D_HID))
    (agg2,) = _sc_seg(z, src, dst)
    out = _tcc(agg2, deg_dst, W2, b2.reshape(1, D_OUT))
    return out


# speedup vs baseline: 19.6941x; 3.1970x over previous
import functools

import jax
import jax.numpy as jnp
from jax import lax
from jax.experimental import pallas as pl
from jax.experimental.pallas import tpu as pltpu
from jax.experimental.pallas import tpu_sc as plsc

N = 10000
E = 320000
D_IN = 128
D_HID = 16
D_OUT = 128

NC = 2
NS = 16
NW = NC * NS
EPW = E // NW
CH = 80
NCH = EPW // CH
K = 25
SB = K * CH
NSB = NCH // K
RPT = N // NS

_mesh = plsc.VectorSubcoreMesh(core_axis_name="c", subcore_axis_name="s")
_sc_params = pltpu.CompilerParams(use_tc_tiling_on_sc=False)


def _fill(buf, nrows, value):
    vec = jnp.full((16,), value, jnp.float32)

    def body(i, _):
        buf[i] = vec
        return 0

    lax.fori_loop(0, nrows, body, 0)


@functools.partial(
    pl.kernel,
    mesh=_mesh,
    compiler_params=_sc_params,
    out_type=jax.ShapeDtypeStruct((NC, N, 16), jnp.float32),
    scratch_types=[
        pltpu.VMEM((NCH, CH), jnp.int32),
        pltpu.VMEM((CH, 16), jnp.float32),
        pltpu.VMEM((RPT, 16), jnp.float32),
        pltpu.VMEM_SHARED((N, 16), jnp.float32),
        pltpu.SemaphoreType.DMA,
    ],
)
def _sc_hist_src(src_hbm, out_hbm, idx_v, ones_v, stage_v, acc_sh, sem_s):
    c = lax.axis_index("c")
    s = lax.axis_index("s")
    wid = s * NC + c
    _fill(stage_v, RPT, 0.0)
    _fill(ones_v, CH, 1.0)
    pltpu.sync_copy(stage_v, acc_sh.at[pl.ds(s * RPT, RPT)])
    plsc.subcore_barrier()
    pltpu.sync_copy(src_hbm.at[pl.ds(wid * NCH, NCH)], idx_v)

    def fire(j, _):
        pltpu.async_copy(ones_v, acc_sh.at[idx_v.at[j]], sem_s, add=True)
        return 0

    lax.fori_loop(0, NCH, fire, 0)
    for _ in range(NCH * CH // RPT):
        pltpu.make_async_copy(src_hbm.at[pl.ds(0, NCH)], stage_v, sem_s).wait()
    plsc.subcore_barrier()
    pltpu.sync_copy(acc_sh.at[pl.ds(s * RPT, RPT)], stage_v)
    pltpu.sync_copy(stage_v, out_hbm.at[c, pl.ds(s * RPT, RPT)])


def _make_sc_seg(with_hist):
    out_type = [jax.ShapeDtypeStruct((NC, N, 16), jnp.float32)]
    scratch = [
        pltpu.VMEM((NCH, CH), jnp.int32),
        pltpu.VMEM((NCH, CH), jnp.int32),
        pltpu.VMEM((SB, 16), jnp.float32),
        pltpu.VMEM((SB, 16), jnp.float32),
        pltpu.VMEM((RPT, 16), jnp.float32),
        pltpu.VMEM_SHARED((N, 16), jnp.float32),
        pltpu.SemaphoreType.DMA,
        pltpu.SemaphoreType.DMA,
        pltpu.SemaphoreType.DMA,
        pltpu.SemaphoreType.DMA,
    ]
    if with_hist:
        out_type = out_type + [jax.ShapeDtypeStruct((NC, N, 16), jnp.float32)]
        scratch = scratch + [
            pltpu.VMEM((CH, 16), jnp.float32),
            pltpu.VMEM_SHARED((N, 16), jnp.float32),
            pltpu.SemaphoreType.DMA,
            pltpu.SemaphoreType.DMA,
        ]

    @functools.partial(
        pl.kernel,
        mesh=_mesh,
        compiler_params=_sc_params,
        out_type=out_type,
        scratch_types=scratch,
    )
    def _seg(table_hbm, src_hbm, dst_hbm, *refs):
        if with_hist:
            (out_hbm, hist_hbm, sidx, didx, rows_a, rows_b, stage_v, acc_sh,
             sem_ga, sem_gb, sem_sa, sem_sb, ones_v, hacc_sh, sem_ha,
             sem_hb) = refs
        else:
            (out_hbm, sidx, didx, rows_a, rows_b, stage_v, acc_sh,
             sem_ga, sem_gb, sem_sa, sem_sb) = refs
        c = lax.axis_index("c")
        s = lax.axis_index("s")
        wid = s * NC + c
        _fill(stage_v, RPT, 0.0)
        pltpu.sync_copy(stage_v, acc_sh.at[pl.ds(s * RPT, RPT)])
        if with_hist:
            _fill(ones_v, CH, 1.0)
            pltpu.sync_copy(stage_v, hacc_sh.at[pl.ds(s * RPT, RPT)])
        plsc.subcore_barrier()
        pltpu.sync_copy(src_hbm.at[pl.ds(wid * NCH, NCH)], sidx)
        pltpu.sync_copy(dst_hbm.at[pl.ds(wid * NCH, NCH)], didx)

        def fire_g(sb, rows_buf, sem):
            def f(k, _):
                pltpu.async_copy(table_hbm.at[sidx.at[sb * K + k]],
                                 rows_buf.at[pl.ds(k * CH, CH)], sem)
                return 0
            lax.fori_loop(0, K, f, 0)

        def fire_s(sb, rows_buf, sem, sem_h):
            def f(k, _):
                pltpu.async_copy(rows_buf.at[pl.ds(k * CH, CH)],
                                 acc_sh.at[didx.at[sb * K + k]], sem,
                                 add=True)
                if with_hist:
                    pltpu.async_copy(ones_v, hacc_sh.at[didx.at[sb * K + k]],
                                     sem_h, add=True)
                return 0
            lax.fori_loop(0, K, f, 0)

        def drain(sem):
            for _ in range(SB // RPT):
                pltpu.make_async_copy(table_hbm.at[pl.ds(0, RPT)], stage_v,
                                      sem).wait()
            pltpu.make_async_copy(table_hbm.at[pl.ds(0, SB % RPT)],
                                  stage_v.at[pl.ds(0, SB % RPT)], sem).wait()

        bufs = (rows_a, rows_b)
        sems_g = (sem_ga, sem_gb)
        sems_s = (sem_sa, sem_sb)
        sems_h = (sem_ha, sem_hb) if with_hist else (None, None)

        fire_g(0, bufs[0], sems_g[0])
        for sb in range(NSB):
            p, q = sb % 2, (sb + 1) % 2
            if sb >= 1:
                drain(sems_s[q])
                if with_hist:
                    drain(sems_h[q])
            if sb < NSB - 1:
                fire_g(sb + 1, bufs[q], sems_g[q])
            drain(sems_g[p])
            fire_s(sb, bufs[p], sems_s[p], sems_h[p])
        last = (NSB - 1) % 2
        drain(sems_s[last])
        if with_hist:
            drain(sems_h[last])

        plsc.subcore_barrier()
        pltpu.sync_copy(acc_sh.at[pl.ds(s * RPT, RPT)], stage_v)
        pltpu.sync_copy(stage_v, out_hbm.at[c, pl.ds(s * RPT, RPT)])
        if with_hist:
            pltpu.sync_copy(hacc_sh.at[pl.ds(s * RPT, RPT)], stage_v)
            pltpu.sync_copy(stage_v, hist_hbm.at[c, pl.ds(s * RPT, RPT)])

    return _seg


_sc_seg_hist = _make_sc_seg(True)
_sc_seg = _make_sc_seg(False)


RB = 400


def _norm(deg_parts):
    deg = deg_parts[0] + deg_parts[1]
    return lax.rsqrt(jnp.maximum(deg, 1.0))


def _tca_body(ds_ref, feat_ref, w1_ref, out_ref):
    nrm = _norm(ds_ref[...])[:, :1]
    out_ref[...] = jnp.dot(feat_ref[...] * nrm, w1_ref[...],
                           preferred_element_type=jnp.float32)


def _tcb_body(a_ref, dd_ref, ds_ref, b1_ref, out_ref):
    agg = (a_ref[0] + a_ref[1]) * _norm(dd_ref[...])
    z = jnp.maximum(agg + b1_ref[...], 0.0)
    out_ref[...] = z * _norm(ds_ref[...])


def _tcc_body(a_ref, dd_ref, w2_ref, b2_ref, out_ref):
    agg = (a_ref[0] + a_ref[1]) * _norm(dd_ref[...])
    out_ref[...] = jnp.dot(agg, w2_ref[...],
                           preferred_element_type=jnp.float32) + b2_ref[...]


_GRID = N // RB
_bs_parts = pl.BlockSpec((NC, RB, 16), lambda i: (0, i, 0))
_bs_h16 = pl.BlockSpec((RB, 16), lambda i: (i, 0))

_tca = pl.pallas_call(
    _tca_body,
    grid=(_GRID,),
    in_specs=[
        _bs_parts,
        pl.BlockSpec((RB, D_IN), lambda i: (i, 0)),
        pl.BlockSpec((D_IN, D_HID), lambda i: (0, 0)),
    ],
    out_specs=_bs_h16,
    out_shape=jax.ShapeDtypeStruct((N, D_HID), jnp.float32),
)

_tcb = pl.pallas_call(
    _tcb_body,
    grid=(_GRID,),
    in_specs=[
        _bs_parts,
        _bs_parts,
        _bs_parts,
        pl.BlockSpec((1, D_HID), lambda i: (0, 0)),
    ],
    out_specs=_bs_h16,
    out_shape=jax.ShapeDtypeStruct((N, D_HID), jnp.float32),
)

_tcc = pl.pallas_call(
    _tcc_body,
    grid=(_GRID,),
    in_specs=[
        _bs_parts,
        _bs_parts,
        pl.BlockSpec((D_HID, D_OUT), lambda i: (0, 0)),
        pl.BlockSpec((1, D_OUT), lambda i: (0, 0)),
    ],
    out_specs=pl.BlockSpec((RB, D_OUT), lambda i: (i, 0)),
    out_shape=jax.ShapeDtypeStruct((N, D_OUT), jnp.float32),
)


@jax.jit
def kernel(feat, edge_index, W1, b1, W2, b2):
    src = edge_index[0].reshape(NW * NCH, CH)
    dst = edge_index[1].reshape(NW * NCH, CH)
    deg_src = _sc_hist_src(src)
    h1 = _tca(deg_src, feat, W1)
    agg1, deg_dst = _sc_seg_hist(h1, src, dst)
    z = _tcb(agg1, deg_dst, deg_src, b1.reshape(1, D_HID))
    (agg2,) = _sc_seg(z, src, dst)
    out = _tcc(agg2, deg_dst, W2, b2.reshape(1, D_OUT))
    return out


# speedup vs baseline: 28.7633x; 1.4605x over previous
import functools

import jax
import jax.numpy as jnp
from jax import lax
from jax.experimental import pallas as pl
from jax.experimental.pallas import tpu as pltpu
from jax.experimental.pallas import tpu_sc as plsc

N = 10000
E = 320000
D_IN = 128
D_HID = 16
D_OUT = 128

NC = 2
NS = 16
NW = NC * NS
EPW = E // NW
CH = 80
NCH = EPW // CH
K = 25
SB = K * CH
NSB = NCH // K
RPT = N // NS
DPT = 640
NP = NS * DPT

_mesh = plsc.VectorSubcoreMesh(core_axis_name="c", subcore_axis_name="s")
_sc_params = pltpu.CompilerParams(use_tc_tiling_on_sc=False)


def _fill_rows(buf, nrows, value):
    vec = jnp.full((16,), value, jnp.float32)

    def body(i, _):
        buf[i] = vec
        return 0

    lax.fori_loop(0, nrows, body, 0)


def _fill_flat(buf, n, value):
    vec = jnp.full((16,), value, jnp.float32)
    for j in range(n // 16):
        buf[pl.ds(j * 16, 16)] = vec


@functools.partial(
    pl.kernel,
    mesh=_mesh,
    compiler_params=_sc_params,
    out_type=jax.ShapeDtypeStruct((NC, NP), jnp.float32),
    scratch_types=[
        pltpu.VMEM((NCH, CH), jnp.int32),
        pltpu.VMEM((CH,), jnp.float32),
        pltpu.VMEM((DPT,), jnp.float32),
        pltpu.VMEM_SHARED((NP,), jnp.float32),
        pltpu.SemaphoreType.DMA,
    ],
)
def _sc_hist_src(src_hbm, out_hbm, idx_v, ones_v, stage_v, acc_sh, sem_s):
    c = lax.axis_index("c")
    s = lax.axis_index("s")
    wid = s * NC + c
    _fill_flat(stage_v, DPT, 0.0)
    _fill_flat(ones_v, CH, 1.0)
    pltpu.sync_copy(stage_v, acc_sh.at[pl.ds(s * DPT, DPT)])
    plsc.subcore_barrier()
    pltpu.sync_copy(src_hbm.at[pl.ds(wid * NCH, NCH)], idx_v)

    def fire(j, _):
        pltpu.async_copy(ones_v, acc_sh.at[idx_v.at[j]], sem_s, add=True)
        return 0

    lax.fori_loop(0, NCH, fire, 0)
    for _ in range(EPW // DPT):
        pltpu.make_async_copy(src_hbm.at[pl.ds(0, DPT // CH)], stage_v,
                              sem_s).wait()
    pltpu.make_async_copy(src_hbm.at[pl.ds(0, (EPW % DPT) // CH)],
                          stage_v.at[pl.ds(0, EPW % DPT)], sem_s).wait()
    plsc.subcore_barrier()
    pltpu.sync_copy(acc_sh.at[pl.ds(s * DPT, DPT)], stage_v)
    pltpu.sync_copy(stage_v, out_hbm.at[c, pl.ds(s * DPT, DPT)])


def _make_sc_seg(with_hist):
    out_type = [jax.ShapeDtypeStruct((NC, N, 16), jnp.float32)]
    scratch = [
        pltpu.VMEM((NCH, CH), jnp.int32),
        pltpu.VMEM((NCH, CH), jnp.int32),
        pltpu.VMEM((SB, 16), jnp.float32),
        pltpu.VMEM((SB, 16), jnp.float32),
        pltpu.VMEM((RPT, 16), jnp.float32),
        pltpu.VMEM_SHARED((N, 16), jnp.float32),
        pltpu.SemaphoreType.DMA,
        pltpu.SemaphoreType.DMA,
        pltpu.SemaphoreType.DMA,
        pltpu.SemaphoreType.DMA,
    ]
    if with_hist:
        out_type = out_type + [jax.ShapeDtypeStruct((NC, NP), jnp.float32)]
        scratch = scratch + [
            pltpu.VMEM((CH,), jnp.float32),
            pltpu.VMEM((DPT,), jnp.float32),
            pltpu.VMEM_SHARED((NP,), jnp.float32),
            pltpu.SemaphoreType.DMA,
        ]

    @functools.partial(
        pl.kernel,
        mesh=_mesh,
        compiler_params=_sc_params,
        out_type=out_type,
        scratch_types=scratch,
    )
    def _seg(table_hbm, src_hbm, dst_hbm, *refs):
        if with_hist:
            (out_hbm, hist_hbm, sidx, didx, rows_a, rows_b, stage_v, acc_sh,
             sem_ga, sem_gb, sem_sa, sem_sb, ones_v, hstage_v, hacc_sh,
             sem_h) = refs
        else:
            (out_hbm, sidx, didx, rows_a, rows_b, stage_v, acc_sh,
             sem_ga, sem_gb, sem_sa, sem_sb) = refs
        c = lax.axis_index("c")
        s = lax.axis_index("s")
        wid = s * NC + c
        _fill_rows(stage_v, RPT, 0.0)
        pltpu.sync_copy(stage_v, acc_sh.at[pl.ds(s * RPT, RPT)])
        if with_hist:
            _fill_flat(ones_v, CH, 1.0)
            _fill_flat(hstage_v, DPT, 0.0)
            pltpu.sync_copy(hstage_v, hacc_sh.at[pl.ds(s * DPT, DPT)])
        plsc.subcore_barrier()
        pltpu.sync_copy(src_hbm.at[pl.ds(wid * NCH, NCH)], sidx)
        pltpu.sync_copy(dst_hbm.at[pl.ds(wid * NCH, NCH)], didx)

        def fire_g(sb, rows_buf, sem):
            def f(k, _):
                pltpu.async_copy(table_hbm.at[sidx.at[sb * K + k]],
                                 rows_buf.at[pl.ds(k * CH, CH)], sem)
                return 0
            lax.fori_loop(0, K, f, 0)

        def fire_s(sb, rows_buf, sem):
            def f(k, _):
                pltpu.async_copy(rows_buf.at[pl.ds(k * CH, CH)],
                                 acc_sh.at[didx.at[sb * K + k]], sem,
                                 add=True)
                if with_hist:
                    pltpu.async_copy(ones_v, hacc_sh.at[didx.at[sb * K + k]],
                                     sem_h, add=True)
                return 0
            lax.fori_loop(0, K, f, 0)

        def drain(sem):
            for _ in range(SB // RPT):
                pltpu.make_async_copy(table_hbm.at[pl.ds(0, RPT)], stage_v,
                                      sem).wait()
            pltpu.make_async_copy(table_hbm.at[pl.ds(0, SB % RPT)],
                                  stage_v.at[pl.ds(0, SB % RPT)], sem).wait()

        bufs = (rows_a, rows_b)
        sems_g = (sem_ga, sem_gb)
        sems_s = (sem_sa, sem_sb)

        fire_g(0, bufs[0], sems_g[0])
        for sb in range(NSB):
            p, q = sb % 2, (sb + 1) % 2
            if sb >= 1:
                drain(sems_s[q])
            if sb < NSB - 1:
                fire_g(sb + 1, bufs[q], sems_g[q])
            drain(sems_g[p])
            fire_s(sb, bufs[p], sems_s[p])
        drain(sems_s[(NSB - 1) % 2])
        if with_hist:
            for _ in range(EPW // DPT):
                pltpu.make_async_copy(src_hbm.at[pl.ds(0, DPT // CH)],
                                      hstage_v, sem_h).wait()
            pltpu.make_async_copy(src_hbm.at[pl.ds(0, (EPW % DPT) // CH)],
                                  hstage_v.at[pl.ds(0, EPW % DPT)],
                                  sem_h).wait()

        plsc.subcore_barrier()
        pltpu.sync_copy(acc_sh.at[pl.ds(s * RPT, RPT)], stage_v)
        pltpu.sync_copy(stage_v, out_hbm.at[c, pl.ds(s * RPT, RPT)])
        if with_hist:
            pltpu.sync_copy(hacc_sh.at[pl.ds(s * DPT, DPT)], hstage_v)
            pltpu.sync_copy(hstage_v, hist_hbm.at[c, pl.ds(s * DPT, DPT)])

    return _seg


_sc_seg_hist = _make_sc_seg(True)
_sc_seg = _make_sc_seg(False)


def _norm(deg_parts):
    deg = (deg_parts[0, :N] + deg_parts[1, :N]).reshape(N, 1)
    return lax.rsqrt(jnp.maximum(deg, 1.0))


def _tca_body(ds_ref, feat_ref, w1_ref, out_ref):
    nrm = _norm(ds_ref[...])
    out_ref[...] = jnp.dot(feat_ref[...] * nrm, w1_ref[...],
                           preferred_element_type=jnp.float32)


def _tcb_body(a_ref, dd_ref, ds_ref, b1_ref, out_ref):
    agg = (a_ref[0] + a_ref[1]) * _norm(dd_ref[...])
    z = jnp.maximum(agg + b1_ref[...], 0.0)
    out_ref[...] = z * _norm(ds_ref[...])


def _tcc_body(a_ref, dd_ref, w2_ref, b2_ref, out_ref):
    agg = (a_ref[0] + a_ref[1]) * _norm(dd_ref[...])
    out_ref[...] = jnp.dot(agg, w2_ref[...],
                           preferred_element_type=jnp.float32) + b2_ref[...]


_tca = pl.pallas_call(
    _tca_body,
    out_shape=jax.ShapeDtypeStruct((N, D_HID), jnp.float32),
)

_tcb = pl.pallas_call(
    _tcb_body,
    out_shape=jax.ShapeDtypeStruct((N, D_HID), jnp.float32),
)

_tcc = pl.pallas_call(
    _tcc_body,
    out_shape=jax.ShapeDtypeStruct((N, D_OUT), jnp.float32),
)


@jax.jit
def kernel(feat, edge_index, W1, b1, W2, b2):
    src = edge_index[0].reshape(NW * NCH, CH)
    dst = edge_index[1].reshape(NW * NCH, CH)
    deg_src = _sc_hist_src(src)
    h1 = _tca(deg_src, feat, W1)
    agg1, deg_dst = _sc_seg_hist(h1, src, dst)
    z = _tcb(agg1, deg_dst, deg_src, b1.reshape(1, D_HID))
    (agg2,) = _sc_seg(z, src, dst)
    out = _tcc(agg2, deg_dst, W2, b2.reshape(1, D_OUT))
    return out
